# baseline (speedup 1.0000x reference)
import jax
import jax.numpy as jnp
from jax import lax
from jax.experimental import pallas as pl
from jax.experimental.pallas import tpu as pltpu
from jax.experimental.pallas import tpu_sc as plsc

B = 8
NN = 32
D = 5
C = 32
TOT = C * C * C
NC = 2
NS = 16
NW = NC * NS
COLS = TOT // NW
SHIFTS = (14, 9, 4)

QN = 2
NQ = NN // QN


def _weights_body(dps_ref, w_ref):
    b = pl.program_id(0)
    q = pl.program_id(1)

    blk = dps_ref[0]

    def cp(v):
        c_iota = jax.lax.broadcasted_iota(jnp.int32, (NN, C), 1)
        acc = jnp.zeros((NN, C), jnp.float32)
        for dd in range(D):
            lo = blk[v * 2 * D + 2 * dd]
            hi = blk[v * 2 * D + 2 * dd + 1]
            diff = hi - lo
            bit = ((c_iota >> (D - 1 - dd)) & 1).astype(jnp.float32)
            acc = acc + lo[:, None] + bit * diff[:, None]
        return jnp.where(q == 0, acc[:NQ], acc[NQ:])

    cp0 = cp(0)
    cp1 = cp(1)
    cp2 = cp(2)
    p12 = cp1[:, None, :, None] + cp2[:, None, None, :]
    w_ref[0] = cp0[:, :, None, None] + p12


def _ct_body(ct0_hbm, ct1_hbm, ct2_hbm, buf, sem):
    cid = lax.axis_index("c")
    sid = lax.axis_index("s")
    wid = (sid * NC + cid).astype(jnp.int32)
    base = wid * COLS
    lane = lax.broadcasted_iota(jnp.int32, (16,), 0)

    for v in range(3):
        for dd in range(D):
            sh = SHIFTS[v] - dd

            @plsc.parallel_loop(0, COLS // 16, unroll=8)
            def _(j, _r=v * D + dd, _sh=sh):
                t = base + j * 16 + lane
                buf[_r, pl.ds(j * 16, 16)] = (t >> _sh) & 1

    dmas = []
    for v, ct_hbm in enumerate((ct0_hbm, ct1_hbm, ct2_hbm)):
        for dd in range(D):
            dmas.append(pltpu.async_copy(
                buf.at[pl.ds(v * D + dd, 1), :],
                ct_hbm.at[pl.ds(dd, 1), pl.ds(base, COLS)], sem))
    for dma in dmas:
        dma.wait()


def kernel(dp0, dp1, dp2):
    dps = jnp.stack([dp0, dp1, dp2], axis=1)
    dps = dps.transpose(0, 1, 2, 4, 3).reshape(B, 3 * D * 2, NN)
    dp_spec = pl.BlockSpec((1, 3 * D * 2, NN), lambda b, q: (b, 0, 0))
    w = pl.pallas_call(
        _weights_body,
        grid=(B, QN),
        in_specs=[dp_spec],
        out_specs=[pl.BlockSpec((1, NQ, C, C, C), lambda b, q: (b, q, 0, 0, 0))],
        out_shape=[jax.ShapeDtypeStruct((B, NN, C, C, C), jnp.float32)],
    )(dps)[0]

    mesh = plsc.VectorSubcoreMesh(
        core_axis_name="c", subcore_axis_name="s", num_cores=NC, num_subcores=NS)
    ct_shape = jax.ShapeDtypeStruct((D, TOT), jnp.int32)
    sc_ct = pl.kernel(
        _ct_body,
        out_type=[ct_shape, ct_shape, ct_shape],
        mesh=mesh,
        compiler_params=pltpu.CompilerParams(needs_layout_passes=False),
        scratch_types=[
            pltpu.VMEM((3 * D, COLS), jnp.int32),
            pltpu.SemaphoreType.DMA,
        ],
    )
    ct0, ct1, ct2 = sc_ct()
    return ct0, ct1, ct2, w


# speedup vs baseline: 1.3216x; 1.3216x over previous
import jax
import jax.numpy as jnp
from jax.experimental import pallas as pl

B = 8
NN = 32
D = 5
C = 32
TOT = C * C * C


QN = 2
NQ = NN // QN


def _weights_body(dps_ref, ct0_ref, ct1_ref, ct2_ref, w_ref):
    b = pl.program_id(0)
    q = pl.program_id(1)

    blk = dps_ref[0]

    def cp(v):
        c_iota = jax.lax.broadcasted_iota(jnp.int32, (NN, C), 1)
        acc = jnp.zeros((NN, C), jnp.float32)
        for dd in range(D):
            lo = blk[v * 2 * D + 2 * dd]
            hi = blk[v * 2 * D + 2 * dd + 1]
            diff = hi - lo
            bit = ((c_iota >> (D - 1 - dd)) & 1).astype(jnp.float32)
            acc = acc + lo[:, None] + bit * diff[:, None]
        return jnp.where(q == 0, acc[:NQ], acc[NQ:])

    cp0 = cp(0)
    cp1 = cp(1)
    cp2 = cp(2)
    p12 = cp1[:, None, :, None] + cp2[:, None, None, :]
    w_ref[0] = cp0[:, :, None, None] + p12

    @pl.when((b == 0) & (q == 0))
    def _():
        t = jax.lax.broadcasted_iota(jnp.int32, (D, TOT), 1)
        d = jax.lax.broadcasted_iota(jnp.int32, (D, TOT), 0)
        ct0_ref[...] = (t >> (14 - d)) & 1
        ct1_ref[...] = (t >> (9 - d)) & 1
        ct2_ref[...] = (t >> (4 - d)) & 1


def kernel(dp0, dp1, dp2):
    dps = jnp.stack([dp0, dp1, dp2], axis=1)
    dps = dps.transpose(0, 1, 2, 4, 3).reshape(B, 3 * D * 2, NN)
    dp_spec = pl.BlockSpec((1, 3 * D * 2, NN), lambda b, q: (b, 0, 0))
    ct_spec = pl.BlockSpec((D, TOT), lambda b, q: (0, 0))
    out = pl.pallas_call(
        _weights_body,
        grid=(B, QN),
        in_specs=[dp_spec],
        out_specs=[
            ct_spec,
            ct_spec,
            ct_spec,
            pl.BlockSpec((1, NQ, C, C, C), lambda b, q: (b, q, 0, 0, 0)),
        ],
        out_shape=[
            jax.ShapeDtypeStruct((D, TOT), jnp.int32),
            jax.ShapeDtypeStruct((D, TOT), jnp.int32),
            jax.ShapeDtypeStruct((D, TOT), jnp.int32),
            jax.ShapeDtypeStruct((B, NN, C, C, C), jnp.float32),
        ],
    )(dps)
    return tuple(out)
